# baseline (speedup 1.0000x reference)
import dataclasses
import functools

import jax
import jax.numpy as jnp
from jax import lax
from jax.experimental import pallas as pl
from jax.experimental.pallas import tpu as pltpu
from jax.experimental.pallas import tpu_sc as plsc

_N = 10000
_E = 320000
_NSUB = 16
_NW = 32
_EPT = _E // _NW
_CH = 80
_NCH = _EPT // _CH
_NP = 10240
_RPT = _NP // _NSUB

_BLK = 1000


def _sc_mesh():
    return plsc.VectorSubcoreMesh(core_axis_name="c", subcore_axis_name="s")


def _sc_degree(pk3):
    cp = pltpu.CompilerParams()
    if "needs_layout_passes" in pltpu.CompilerParams.__dataclass_fields__:
        cp = dataclasses.replace(cp, needs_layout_passes=False)

    @functools.partial(
        pl.kernel,
        out_type=jax.ShapeDtypeStruct((2, _NP), jnp.float32),
        mesh=_sc_mesh(),
        compiler_params=cp,
        scratch_types=[
            pltpu.VMEM((_EPT,), jnp.int32),
            pltpu.VMEM((_NP,), jnp.float32),
            pltpu.VMEM((_NSUB, _RPT), jnp.float32),
            pltpu.VMEM((_RPT,), jnp.float32),
            pltpu.VMEM_SHARED((_NSUB, _NP), jnp.float32),
        ],
    )
    def deg_kernel(pk_hbm, out_hbm, pk_v, tab_v, red_v, sum_v, stage_sh):
        c = lax.axis_index("c")
        s = lax.axis_index("s")
        wid = c * _NSUB + s
        pltpu.sync_copy(pk_hbm.at[wid], pk_v)

        @pl.loop(0, _NP // 16, unroll=8)
        def _(t):
            tab_v[pl.ds(t * 16, 16)] = jnp.zeros((16,), jnp.float32)

        ones = jnp.ones((16,), jnp.float32)

        @pl.loop(0, _EPT // 16, unroll=8)
        def _(n):
            ci = lax.shift_right_logical(pk_v[pl.ds(n * 16, 16)], 16)
            plsc.addupdate_scatter(tab_v, [ci], ones)

        pltpu.sync_copy(tab_v, stage_sh.at[s])
        plsc.subcore_barrier()
        pltpu.sync_copy(stage_sh.at[:, pl.ds(s * _RPT, _RPT)], red_v)

        @pl.loop(0, _RPT // 16, unroll=4)
        def _(q):
            acc = red_v[0, pl.ds(q * 16, 16)]
            for r in range(1, _NSUB):
                acc = acc + red_v[r, pl.ds(q * 16, 16)]
            sum_v[pl.ds(q * 16, 16)] = acc

        pltpu.sync_copy(sum_v, out_hbm.at[c, pl.ds(s * _RPT, _RPT)])

    return deg_kernel(pk3)


def _sc_aggregate(p, pk3):
    zeros = jnp.zeros((_RPT, 128), jnp.float32)

    @functools.partial(
        pl.kernel,
        out_type=jax.ShapeDtypeStruct((2, _NP, 128), jnp.float32),
        mesh=_sc_mesh(),
        scratch_types=[
            pltpu.VMEM((_EPT,), jnp.int32),
            pltpu.VMEM((_CH,), jnp.int32),
            pltpu.VMEM((_CH,), jnp.int32),
            pltpu.VMEM((_CH,), jnp.int32),
            pltpu.VMEM((_CH,), jnp.int32),
            pltpu.VMEM((_CH, 128), jnp.float32),
            pltpu.VMEM((_CH, 128), jnp.float32),
            pltpu.VMEM_SHARED((_NP, 128), jnp.float32),
            pltpu.SemaphoreType.DMA,
            pltpu.SemaphoreType.DMA,
        ],
    )
    def agg_kernel(p_hbm, pk_hbm, zeros_hbm, out_hbm,
                   pk_v, ri0_v, ci0_v, ri1_v, ci1_v, rows0_v, rows1_v,
                   agg_sh, sem0, sem1):
        c = lax.axis_index("c")
        s = lax.axis_index("s")
        wid = c * _NSUB + s
        pltpu.sync_copy(pk_hbm.at[wid], pk_v)
        pltpu.sync_copy(zeros_hbm, agg_sh.at[pl.ds(s * _RPT, _RPT)])

        def unpack(m, ri_b, ci_b):
            @pl.loop(0, _CH // 16)
            def _(t):
                v = pk_v[pl.ds(m * _CH + t * 16, 16)]
                ri_b[pl.ds(t * 16, 16)] = lax.bitwise_and(v, 0xFFFF)
                ci_b[pl.ds(t * 16, 16)] = lax.shift_right_logical(v, 16)

        unpack(0, ri0_v, ci0_v)
        unpack(1, ri1_v, ci1_v)
        plsc.subcore_barrier()

        pltpu.async_copy(p_hbm.at[ri0_v], rows0_v, sem0)

        @pl.loop(0, (_NCH - 1) // 2)
        def _(k):
            j1 = 2 * k + 1
            pltpu.make_async_copy(p_hbm.at[ri0_v], rows0_v, sem0).wait()
            pltpu.async_copy(p_hbm.at[ri1_v], rows1_v, sem1)
            pltpu.sync_copy(rows0_v, agg_sh.at[ci0_v], add=True)
            unpack(j1 + 1, ri0_v, ci0_v)
            pltpu.make_async_copy(p_hbm.at[ri1_v], rows1_v, sem1).wait()
            pltpu.async_copy(p_hbm.at[ri0_v], rows0_v, sem0)
            pltpu.sync_copy(rows1_v, agg_sh.at[ci1_v], add=True)

            @pl.when(j1 + 2 < _NCH)
            def _():
                unpack(j1 + 2, ri1_v, ci1_v)

        pltpu.make_async_copy(p_hbm.at[ri0_v], rows0_v, sem0).wait()
        pltpu.sync_copy(rows0_v, agg_sh.at[ci0_v], add=True)

        plsc.subcore_barrier()
        pltpu.sync_copy(agg_sh.at[pl.ds(s * _RPT, _RPT)],
                        out_hbm.at[c, pl.ds(s * _RPT, _RPT)])

    return agg_kernel(p, pk3, zeros)


def _tc_encode_scale(x, w1, b1, w2, b2, g, b, deg0, deg1):
    def body(x_ref, w1_ref, b1_ref, w2_ref, b2_ref, g_ref, bb_ref,
             d0_ref, d1_ref, h_ref, p_ref):
        h = jnp.maximum(
            jnp.dot(x_ref[...], w1_ref[...],
                    preferred_element_type=jnp.float32,
                    precision=lax.Precision.HIGHEST) + b1_ref[...], 0.0)
        h = jnp.dot(h, w2_ref[...],
                    preferred_element_type=jnp.float32,
                    precision=lax.Precision.HIGHEST) + b2_ref[...]
        mu = jnp.mean(h, axis=-1, keepdims=True)
        var = jnp.mean((h - mu) ** 2, axis=-1, keepdims=True)
        h = (h - mu) * lax.rsqrt(var + 1e-5) * g_ref[...] + bb_ref[...]
        h_ref[...] = h
        d = d0_ref[...] + d1_ref[...] + 2.0
        p_ref[...] = lax.rsqrt(d) * h

    full = lambda shape: pl.BlockSpec(shape, lambda i: (0, 0))
    return pl.pallas_call(
        body,
        grid=(_N // _BLK,),
        in_specs=[
            pl.BlockSpec((_BLK, 128), lambda i: (i, 0)),
            full((128, 128)), full((1, 128)),
            full((128, 128)), full((1, 128)),
            full((1, 128)), full((1, 128)),
            pl.BlockSpec((_BLK, 1), lambda i: (i, 0)),
            pl.BlockSpec((_BLK, 1), lambda i: (i, 0)),
        ],
        out_specs=[pl.BlockSpec((_BLK, 128), lambda i: (i, 0)),
                   pl.BlockSpec((_BLK, 128), lambda i: (i, 0))],
        out_shape=[jax.ShapeDtypeStruct((_N, 128), jnp.float32),
                   jax.ShapeDtypeStruct((_N, 128), jnp.float32)],
    )(x, w1, b1, w2, b2, g, b, deg0, deg1)


def _tc_decode(h, agg0, agg1, deg0, deg1, conv_W, conv_b, skip_W, skip_b,
               dec_W1, dec_b1, dec_W2, dec_b2):
    def body(h_ref, a0_ref, a1_ref, d0_ref, d1_ref, cw_ref, cb_ref,
             sw_ref, sb_ref, w1_ref, b1_ref, w2_ref, b2_ref, y_ref):
        h = h_ref[...]
        d = d0_ref[...] + d1_ref[...] + 2.0
        dinv = lax.rsqrt(d)
        tmp = dinv * (a0_ref[...] + a1_ref[...]) + (2.0 * dinv * dinv) * h
        out = (jnp.dot(tmp, cw_ref[...], preferred_element_type=jnp.float32,
                    precision=lax.Precision.HIGHEST)
               + cb_ref[...]
               + jnp.dot(h, sw_ref[...], preferred_element_type=jnp.float32,
                    precision=lax.Precision.HIGHEST)
               + sb_ref[...])
        out = jnp.where(out > 0, out, 0.1 * (jnp.exp(out) - 1.0))
        dd = jnp.dot(out, w1_ref[...],
                     preferred_element_type=jnp.float32,
                    precision=lax.Precision.HIGHEST) + b1_ref[...]
        dd = jnp.where(dd > 0, dd, 0.1 * dd)
        y_ref[...] = jnp.dot(dd, w2_ref[...],
                             preferred_element_type=jnp.float32,
                    precision=lax.Precision.HIGHEST) + b2_ref[...]

    full = lambda shape: pl.BlockSpec(shape, lambda i: (0, 0))
    return pl.pallas_call(
        body,
        grid=(_N // _BLK,),
        in_specs=[
            pl.BlockSpec((_BLK, 128), lambda i: (i, 0)),
            pl.BlockSpec((_BLK, 128), lambda i: (i, 0)),
            pl.BlockSpec((_BLK, 128), lambda i: (i, 0)),
            pl.BlockSpec((_BLK, 1), lambda i: (i, 0)),
            pl.BlockSpec((_BLK, 1), lambda i: (i, 0)),
            full((128, 256)), full((1, 256)),
            full((128, 256)), full((1, 256)),
            full((256, 256)), full((1, 256)),
            full((256, 1)), full((1, 1)),
        ],
        out_specs=pl.BlockSpec((_BLK, 1), lambda i: (i, 0)),
        out_shape=jax.ShapeDtypeStruct((_N, 1), jnp.float32),
    )(h, agg0, agg1, deg0, deg1, conv_W, conv_b, skip_W, skip_b,
      dec_W1, dec_b1, dec_W2, dec_b2)


def kernel(x, edge_index, enc_W1, enc_b1, enc_W2, enc_b2, ln_g, ln_b,
           conv_W, conv_b, skip_W, skip_b, dec_W1, dec_b1, dec_W2, dec_b2):
    row = edge_index[0]
    col = edge_index[1]
    pk3 = (row | (col << 16)).reshape(_NW, _EPT)

    deg_parts = _sc_degree(pk3)
    deg0 = deg_parts[0].reshape(_NP, 1)
    deg1 = deg_parts[1].reshape(_NP, 1)
    h, p = _tc_encode_scale(x, enc_W1, enc_b1.reshape(1, -1), enc_W2,
                            enc_b2.reshape(1, -1), ln_g.reshape(1, -1),
                            ln_b.reshape(1, -1), deg0, deg1)
    agg_parts = _sc_aggregate(p, pk3)
    return _tc_decode(h, agg_parts[0], agg_parts[1], deg0, deg1,
                      conv_W, conv_b.reshape(1, -1), skip_W,
                      skip_b.reshape(1, -1), dec_W1, dec_b1.reshape(1, -1),
                      dec_W2, dec_b2.reshape(1, -1))


# speedup vs baseline: 1.0037x; 1.0037x over previous
import dataclasses
import functools

import jax
import jax.numpy as jnp
from jax import lax
from jax.experimental import pallas as pl
from jax.experimental.pallas import tpu as pltpu
from jax.experimental.pallas import tpu_sc as plsc

_N = 10000
_E = 320000
_NSUB = 16
_NW = 32
_EPT = _E // _NW
_CH = 80
_NCH = _EPT // _CH
_NP = 10240
_RPT = _NP // _NSUB

_BLK = 1000


def _sc_mesh():
    return plsc.VectorSubcoreMesh(core_axis_name="c", subcore_axis_name="s")


def _sc_degree(pk3):
    cp = pltpu.CompilerParams()
    if "needs_layout_passes" in pltpu.CompilerParams.__dataclass_fields__:
        cp = dataclasses.replace(cp, needs_layout_passes=False)

    @functools.partial(
        pl.kernel,
        out_type=jax.ShapeDtypeStruct((2, _NP), jnp.float32),
        mesh=_sc_mesh(),
        compiler_params=cp,
        scratch_types=[
            pltpu.VMEM((_EPT,), jnp.int32),
            pltpu.VMEM((_NP,), jnp.float32),
            pltpu.VMEM((_NSUB, _RPT), jnp.float32),
            pltpu.VMEM((_RPT,), jnp.float32),
            pltpu.VMEM_SHARED((_NSUB, _NP), jnp.float32),
        ],
    )
    def deg_kernel(pk_hbm, out_hbm, pk_v, tab_v, red_v, sum_v, stage_sh):
        c = lax.axis_index("c")
        s = lax.axis_index("s")
        wid = c * _NSUB + s
        pltpu.sync_copy(pk_hbm.at[wid], pk_v)

        @pl.loop(0, _NP // 16, unroll=8)
        def _(t):
            tab_v[pl.ds(t * 16, 16)] = jnp.zeros((16,), jnp.float32)

        ones = jnp.ones((16,), jnp.float32)

        @pl.loop(0, _EPT // 16, unroll=8)
        def _(n):
            ci = lax.shift_right_logical(pk_v[pl.ds(n * 16, 16)], 16)
            plsc.addupdate_scatter(tab_v, [ci], ones)

        pltpu.sync_copy(tab_v, stage_sh.at[s])
        plsc.subcore_barrier()
        pltpu.sync_copy(stage_sh.at[:, pl.ds(s * _RPT, _RPT)], red_v)

        @pl.loop(0, _RPT // 16, unroll=4)
        def _(q):
            acc = red_v[0, pl.ds(q * 16, 16)]
            for r in range(1, _NSUB):
                acc = acc + red_v[r, pl.ds(q * 16, 16)]
            sum_v[pl.ds(q * 16, 16)] = acc

        pltpu.sync_copy(sum_v, out_hbm.at[c, pl.ds(s * _RPT, _RPT)])

    return deg_kernel(pk3)


def _sc_aggregate(p, pk3):
    zeros = jnp.zeros((_RPT, 128), jnp.float32)

    @functools.partial(
        pl.kernel,
        out_type=jax.ShapeDtypeStruct((2, _NP, 128), jnp.float32),
        mesh=_sc_mesh(),
        scratch_types=[
            pltpu.VMEM((_EPT,), jnp.int32),
            pltpu.VMEM((_CH,), jnp.int32),
            pltpu.VMEM((_CH,), jnp.int32),
            pltpu.VMEM((_CH,), jnp.int32),
            pltpu.VMEM((_CH,), jnp.int32),
            pltpu.VMEM((_CH, 128), jnp.float32),
            pltpu.VMEM((_CH, 128), jnp.float32),
            pltpu.VMEM_SHARED((_NP, 128), jnp.float32),
            pltpu.SemaphoreType.DMA,
            pltpu.SemaphoreType.DMA,
            pltpu.SemaphoreType.DMA,
            pltpu.SemaphoreType.DMA,
        ],
    )
    def agg_kernel(p_hbm, pk_hbm, zeros_hbm, out_hbm,
                   pk_v, ri0_v, ci0_v, ri1_v, ci1_v, rows0_v, rows1_v,
                   agg_sh, gsem0, gsem1, ssem0, ssem1):
        c = lax.axis_index("c")
        s = lax.axis_index("s")
        wid = c * _NSUB + s
        pltpu.sync_copy(pk_hbm.at[wid], pk_v)
        pltpu.sync_copy(zeros_hbm, agg_sh.at[pl.ds(s * _RPT, _RPT)])

        def unpack(m, ri_b, ci_b):
            @pl.loop(0, _CH // 16)
            def _(t):
                v = pk_v[pl.ds(m * _CH + t * 16, 16)]
                ri_b[pl.ds(t * 16, 16)] = lax.bitwise_and(v, 0xFFFF)
                ci_b[pl.ds(t * 16, 16)] = lax.shift_right_logical(v, 16)

        unpack(0, ri0_v, ci0_v)
        unpack(1, ri1_v, ci1_v)
        plsc.subcore_barrier()

        pltpu.async_copy(p_hbm.at[ri0_v], rows0_v, gsem0)
        pltpu.async_copy(p_hbm.at[ri1_v], rows1_v, gsem1)

        @pl.loop(0, (_NCH - 1) // 2)
        def _(k):
            j1 = 2 * k + 1
            pltpu.make_async_copy(p_hbm.at[ri0_v], rows0_v, gsem0).wait()
            pltpu.async_copy(rows0_v, agg_sh.at[ci0_v], ssem0, add=True)
            pltpu.make_async_copy(p_hbm.at[ri1_v], rows1_v, gsem1).wait()
            pltpu.async_copy(rows1_v, agg_sh.at[ci1_v], ssem1, add=True)
            pltpu.make_async_copy(rows0_v, agg_sh.at[ci0_v], ssem0).wait()
            unpack(j1 + 1, ri0_v, ci0_v)
            pltpu.async_copy(p_hbm.at[ri0_v], rows0_v, gsem0)
            pltpu.make_async_copy(rows1_v, agg_sh.at[ci1_v], ssem1).wait()

            @pl.when(j1 + 2 < _NCH)
            def _():
                unpack(j1 + 2, ri1_v, ci1_v)
                pltpu.async_copy(p_hbm.at[ri1_v], rows1_v, gsem1)

        pltpu.make_async_copy(p_hbm.at[ri0_v], rows0_v, gsem0).wait()
        pltpu.sync_copy(rows0_v, agg_sh.at[ci0_v], add=True)

        plsc.subcore_barrier()
        pltpu.sync_copy(agg_sh.at[pl.ds(s * _RPT, _RPT)],
                        out_hbm.at[c, pl.ds(s * _RPT, _RPT)])

    return agg_kernel(p, pk3, zeros)


def _tc_encode_scale(x, w1, b1, w2, b2, g, b, deg0, deg1):
    def body(x_ref, w1_ref, b1_ref, w2_ref, b2_ref, g_ref, bb_ref,
             d0_ref, d1_ref, h_ref, p_ref):
        h = jnp.maximum(
            jnp.dot(x_ref[...], w1_ref[...],
                    preferred_element_type=jnp.float32,
                    precision=lax.Precision.HIGHEST) + b1_ref[...], 0.0)
        h = jnp.dot(h, w2_ref[...],
                    preferred_element_type=jnp.float32,
                    precision=lax.Precision.HIGHEST) + b2_ref[...]
        mu = jnp.mean(h, axis=-1, keepdims=True)
        var = jnp.mean((h - mu) ** 2, axis=-1, keepdims=True)
        h = (h - mu) * lax.rsqrt(var + 1e-5) * g_ref[...] + bb_ref[...]
        h_ref[...] = h
        d = d0_ref[...] + d1_ref[...] + 2.0
        p_ref[...] = lax.rsqrt(d) * h

    full = lambda shape: pl.BlockSpec(shape, lambda i: (0, 0))
    return pl.pallas_call(
        body,
        grid=(_N // _BLK,),
        in_specs=[
            pl.BlockSpec((_BLK, 128), lambda i: (i, 0)),
            full((128, 128)), full((1, 128)),
            full((128, 128)), full((1, 128)),
            full((1, 128)), full((1, 128)),
            pl.BlockSpec((_BLK, 1), lambda i: (i, 0)),
            pl.BlockSpec((_BLK, 1), lambda i: (i, 0)),
        ],
        out_specs=[pl.BlockSpec((_BLK, 128), lambda i: (i, 0)),
                   pl.BlockSpec((_BLK, 128), lambda i: (i, 0))],
        out_shape=[jax.ShapeDtypeStruct((_N, 128), jnp.float32),
                   jax.ShapeDtypeStruct((_N, 128), jnp.float32)],
    )(x, w1, b1, w2, b2, g, b, deg0, deg1)


def _tc_decode(h, agg0, agg1, deg0, deg1, conv_W, conv_b, skip_W, skip_b,
               dec_W1, dec_b1, dec_W2, dec_b2):
    def body(h_ref, a0_ref, a1_ref, d0_ref, d1_ref, cw_ref, cb_ref,
             sw_ref, sb_ref, w1_ref, b1_ref, w2_ref, b2_ref, y_ref):
        h = h_ref[...]
        d = d0_ref[...] + d1_ref[...] + 2.0
        dinv = lax.rsqrt(d)
        tmp = dinv * (a0_ref[...] + a1_ref[...]) + (2.0 * dinv * dinv) * h
        out = (jnp.dot(tmp, cw_ref[...], preferred_element_type=jnp.float32,
                    precision=lax.Precision.HIGHEST)
               + cb_ref[...]
               + jnp.dot(h, sw_ref[...], preferred_element_type=jnp.float32,
                    precision=lax.Precision.HIGHEST)
               + sb_ref[...])
        out = jnp.where(out > 0, out, 0.1 * (jnp.exp(out) - 1.0))
        dd = jnp.dot(out, w1_ref[...],
                     preferred_element_type=jnp.float32,
                    precision=lax.Precision.HIGHEST) + b1_ref[...]
        dd = jnp.where(dd > 0, dd, 0.1 * dd)
        y_ref[...] = jnp.dot(dd, w2_ref[...],
                             preferred_element_type=jnp.float32,
                    precision=lax.Precision.HIGHEST) + b2_ref[...]

    full = lambda shape: pl.BlockSpec(shape, lambda i: (0, 0))
    return pl.pallas_call(
        body,
        grid=(_N // _BLK,),
        in_specs=[
            pl.BlockSpec((_BLK, 128), lambda i: (i, 0)),
            pl.BlockSpec((_BLK, 128), lambda i: (i, 0)),
            pl.BlockSpec((_BLK, 128), lambda i: (i, 0)),
            pl.BlockSpec((_BLK, 1), lambda i: (i, 0)),
            pl.BlockSpec((_BLK, 1), lambda i: (i, 0)),
            full((128, 256)), full((1, 256)),
            full((128, 256)), full((1, 256)),
            full((256, 256)), full((1, 256)),
            full((256, 1)), full((1, 1)),
        ],
        out_specs=pl.BlockSpec((_BLK, 1), lambda i: (i, 0)),
        out_shape=jax.ShapeDtypeStruct((_N, 1), jnp.float32),
    )(h, agg0, agg1, deg0, deg1, conv_W, conv_b, skip_W, skip_b,
      dec_W1, dec_b1, dec_W2, dec_b2)


def kernel(x, edge_index, enc_W1, enc_b1, enc_W2, enc_b2, ln_g, ln_b,
           conv_W, conv_b, skip_W, skip_b, dec_W1, dec_b1, dec_W2, dec_b2):
    row = edge_index[0]
    col = edge_index[1]
    pk3 = (row | (col << 16)).reshape(_NW, _EPT)

    deg_parts = _sc_degree(pk3)
    deg0 = deg_parts[0].reshape(_NP, 1)
    deg1 = deg_parts[1].reshape(_NP, 1)
    h, p = _tc_encode_scale(x, enc_W1, enc_b1.reshape(1, -1), enc_W2,
                            enc_b2.reshape(1, -1), ln_g.reshape(1, -1),
                            ln_b.reshape(1, -1), deg0, deg1)
    agg_parts = _sc_aggregate(p, pk3)
    return _tc_decode(h, agg_parts[0], agg_parts[1], deg0, deg1,
                      conv_W, conv_b.reshape(1, -1), skip_W,
                      skip_b.reshape(1, -1), dec_W1, dec_b1.reshape(1, -1),
                      dec_W2, dec_b2.reshape(1, -1))


# speedup vs baseline: 1.1675x; 1.1632x over previous
import dataclasses
import functools

import jax
import jax.numpy as jnp
from jax import lax
from jax.experimental import pallas as pl
from jax.experimental.pallas import tpu as pltpu
from jax.experimental.pallas import tpu_sc as plsc

_N = 10000
_E = 320000
_NSUB = 16
_NW = 32
_EPT = _E // _NW
_CH = 80
_NCH = _EPT // _CH
_NP = 10240
_RPT = _NP // _NSUB

_BLK = 1000


def _sc_mesh():
    return plsc.VectorSubcoreMesh(core_axis_name="c", subcore_axis_name="s")


def _sc_degree(pk3):
    cp = pltpu.CompilerParams()
    if "needs_layout_passes" in pltpu.CompilerParams.__dataclass_fields__:
        cp = dataclasses.replace(cp, needs_layout_passes=False)

    @functools.partial(
        pl.kernel,
        out_type=jax.ShapeDtypeStruct((2, _NP), jnp.float32),
        mesh=_sc_mesh(),
        compiler_params=cp,
        scratch_types=[
            pltpu.VMEM((_EPT,), jnp.int32),
            pltpu.VMEM((_NP,), jnp.float32),
            pltpu.VMEM((_NSUB, _RPT), jnp.float32),
            pltpu.VMEM((_RPT,), jnp.float32),
            pltpu.VMEM_SHARED((_NSUB, _NP), jnp.float32),
        ],
    )
    def deg_kernel(pk_hbm, out_hbm, pk_v, tab_v, red_v, sum_v, stage_sh):
        c = lax.axis_index("c")
        s = lax.axis_index("s")
        wid = c * _NSUB + s
        pltpu.sync_copy(pk_hbm.at[wid], pk_v)

        @pl.loop(0, _NP // 16, unroll=8)
        def _(t):
            tab_v[pl.ds(t * 16, 16)] = jnp.zeros((16,), jnp.float32)

        ones = jnp.ones((16,), jnp.float32)

        @pl.loop(0, _EPT // 16, unroll=8)
        def _(n):
            ci = lax.shift_right_logical(pk_v[pl.ds(n * 16, 16)], 16)
            plsc.addupdate_scatter(tab_v, [ci], ones)

        pltpu.sync_copy(tab_v, stage_sh.at[s])
        plsc.subcore_barrier()
        pltpu.sync_copy(stage_sh.at[:, pl.ds(s * _RPT, _RPT)], red_v)

        @pl.loop(0, _RPT // 16, unroll=4)
        def _(q):
            acc = red_v[0, pl.ds(q * 16, 16)]
            for r in range(1, _NSUB):
                acc = acc + red_v[r, pl.ds(q * 16, 16)]
            sum_v[pl.ds(q * 16, 16)] = acc

        pltpu.sync_copy(sum_v, out_hbm.at[c, pl.ds(s * _RPT, _RPT)])

    return deg_kernel(pk3)


def _sc_aggregate(p, pk3):
    zeros = jnp.zeros((_RPT, 128), jnp.float32)

    @functools.partial(
        pl.kernel,
        out_type=jax.ShapeDtypeStruct((2, _NP, 128), jnp.float32),
        mesh=_sc_mesh(),
        scratch_types=[
            pltpu.VMEM((_EPT,), jnp.int32),
            pltpu.VMEM((_CH,), jnp.int32),
            pltpu.VMEM((_CH,), jnp.int32),
            pltpu.VMEM((_CH,), jnp.int32),
            pltpu.VMEM((_CH,), jnp.int32),
            pltpu.VMEM((_CH, 128), jnp.float32),
            pltpu.VMEM((_CH, 128), jnp.float32),
            pltpu.VMEM_SHARED((_NP, 128), jnp.float32),
            pltpu.SemaphoreType.DMA,
            pltpu.SemaphoreType.DMA,
            pltpu.SemaphoreType.DMA,
            pltpu.SemaphoreType.DMA,
        ],
    )
    def agg_kernel(p_hbm, pk_hbm, zeros_hbm, out_hbm,
                   pk_v, ri0_v, ci0_v, ri1_v, ci1_v, rows0_v, rows1_v,
                   agg_sh, gsem0, gsem1, ssem0, ssem1):
        c = lax.axis_index("c")
        s = lax.axis_index("s")
        wid = c * _NSUB + s
        pltpu.sync_copy(pk_hbm.at[wid], pk_v)
        pltpu.sync_copy(zeros_hbm, agg_sh.at[pl.ds(s * _RPT, _RPT)])

        def unpack(m, ri_b, ci_b):
            @pl.loop(0, _CH // 16)
            def _(t):
                v = pk_v[pl.ds(m * _CH + t * 16, 16)]
                ri_b[pl.ds(t * 16, 16)] = lax.bitwise_and(v, 0xFFFF)
                ci_b[pl.ds(t * 16, 16)] = lax.shift_right_logical(v, 16)

        unpack(0, ri0_v, ci0_v)
        unpack(1, ri1_v, ci1_v)
        plsc.subcore_barrier()

        pltpu.async_copy(p_hbm.at[ri0_v], rows0_v, gsem0)
        pltpu.async_copy(p_hbm.at[ri1_v], rows1_v, gsem1)

        @pl.loop(0, (_NCH - 1) // 2)
        def _(k):
            j1 = 2 * k + 1
            pltpu.make_async_copy(p_hbm.at[ri0_v], rows0_v, gsem0).wait()
            pltpu.async_copy(rows0_v, agg_sh.at[ci0_v], ssem0, add=True)
            pltpu.make_async_copy(p_hbm.at[ri1_v], rows1_v, gsem1).wait()
            pltpu.async_copy(rows1_v, agg_sh.at[ci1_v], ssem1, add=True)
            pltpu.make_async_copy(rows0_v, agg_sh.at[ci0_v], ssem0).wait()
            unpack(j1 + 1, ri0_v, ci0_v)
            pltpu.async_copy(p_hbm.at[ri0_v], rows0_v, gsem0)
            pltpu.make_async_copy(rows1_v, agg_sh.at[ci1_v], ssem1).wait()

            @pl.when(j1 + 2 < _NCH)
            def _():
                unpack(j1 + 2, ri1_v, ci1_v)
                pltpu.async_copy(p_hbm.at[ri1_v], rows1_v, gsem1)

        pltpu.make_async_copy(p_hbm.at[ri0_v], rows0_v, gsem0).wait()
        pltpu.sync_copy(rows0_v, agg_sh.at[ci0_v], add=True)

        plsc.subcore_barrier()
        pltpu.sync_copy(agg_sh.at[pl.ds(s * _RPT, _RPT)],
                        out_hbm.at[c, pl.ds(s * _RPT, _RPT)])

    return agg_kernel(p, pk3, zeros)


def _dot3(x, whi, wlo):
    xhi = x.astype(jnp.bfloat16)
    xlo = (x - xhi.astype(jnp.float32)).astype(jnp.bfloat16)
    return (jnp.dot(xhi, whi, preferred_element_type=jnp.float32)
            + jnp.dot(xhi, wlo, preferred_element_type=jnp.float32)
            + jnp.dot(xlo, whi, preferred_element_type=jnp.float32))


def _split_w(w):
    whi = w.astype(jnp.bfloat16)
    wlo = (w - whi.astype(jnp.float32)).astype(jnp.bfloat16)
    return whi, wlo


def _tc_encode_scale(x, w1hi, w1lo, b1, w2hi, w2lo, b2, g, b, deg0, deg1):
    def body(x_ref, w1hi_ref, w1lo_ref, b1_ref, w2hi_ref, w2lo_ref, b2_ref,
             g_ref, bb_ref, d0_ref, d1_ref, h_ref, p_ref):
        h = jnp.maximum(
            _dot3(x_ref[...], w1hi_ref[...], w1lo_ref[...]) + b1_ref[...], 0.0)
        h = _dot3(h, w2hi_ref[...], w2lo_ref[...]) + b2_ref[...]
        mu = jnp.mean(h, axis=-1, keepdims=True)
        var = jnp.mean((h - mu) ** 2, axis=-1, keepdims=True)
        h = (h - mu) * lax.rsqrt(var + 1e-5) * g_ref[...] + bb_ref[...]
        h_ref[...] = h
        d = d0_ref[...] + d1_ref[...] + 2.0
        p_ref[...] = lax.rsqrt(d) * h

    full = lambda shape: pl.BlockSpec(shape, lambda i: (0, 0))
    return pl.pallas_call(
        body,
        grid=(_N // _BLK,),
        in_specs=[
            pl.BlockSpec((_BLK, 128), lambda i: (i, 0)),
            full((128, 128)), full((128, 128)), full((1, 128)),
            full((128, 128)), full((128, 128)), full((1, 128)),
            full((1, 128)), full((1, 128)),
            pl.BlockSpec((_BLK, 1), lambda i: (i, 0)),
            pl.BlockSpec((_BLK, 1), lambda i: (i, 0)),
        ],
        out_specs=[pl.BlockSpec((_BLK, 128), lambda i: (i, 0)),
                   pl.BlockSpec((_BLK, 128), lambda i: (i, 0))],
        out_shape=[jax.ShapeDtypeStruct((_N, 128), jnp.float32),
                   jax.ShapeDtypeStruct((_N, 128), jnp.float32)],
    )(x, w1hi, w1lo, b1, w2hi, w2lo, b2, g, b, deg0, deg1)


def _tc_decode(h, agg0, agg1, deg0, deg1, cwhi, cwlo, conv_b, swhi, swlo,
               skip_b, w1hi, w1lo, dec_b1, dec_W2, dec_b2):
    def body(h_ref, a0_ref, a1_ref, d0_ref, d1_ref, cwhi_ref, cwlo_ref,
             cb_ref, swhi_ref, swlo_ref, sb_ref, w1hi_ref, w1lo_ref, b1_ref,
             w2_ref, b2_ref, y_ref):
        h = h_ref[...]
        d = d0_ref[...] + d1_ref[...] + 2.0
        dinv = lax.rsqrt(d)
        tmp = dinv * (a0_ref[...] + a1_ref[...]) + (2.0 * dinv * dinv) * h
        out = (_dot3(tmp, cwhi_ref[...], cwlo_ref[...]) + cb_ref[...]
               + _dot3(h, swhi_ref[...], swlo_ref[...]) + sb_ref[...])
        out = jnp.where(out > 0, out, 0.1 * (jnp.exp(out) - 1.0))
        dd = _dot3(out, w1hi_ref[...], w1lo_ref[...]) + b1_ref[...]
        dd = jnp.where(dd > 0, dd, 0.1 * dd)
        y_ref[...] = jnp.dot(dd, w2_ref[...],
                             preferred_element_type=jnp.float32,
                    precision=lax.Precision.HIGHEST) + b2_ref[...]

    full = lambda shape: pl.BlockSpec(shape, lambda i: (0, 0))
    return pl.pallas_call(
        body,
        grid=(_N // _BLK,),
        in_specs=[
            pl.BlockSpec((_BLK, 128), lambda i: (i, 0)),
            pl.BlockSpec((_BLK, 128), lambda i: (i, 0)),
            pl.BlockSpec((_BLK, 128), lambda i: (i, 0)),
            pl.BlockSpec((_BLK, 1), lambda i: (i, 0)),
            pl.BlockSpec((_BLK, 1), lambda i: (i, 0)),
            full((128, 256)), full((128, 256)), full((1, 256)),
            full((128, 256)), full((128, 256)), full((1, 256)),
            full((256, 256)), full((256, 256)), full((1, 256)),
            full((256, 1)), full((1, 1)),
        ],
        out_specs=pl.BlockSpec((_BLK, 1), lambda i: (i, 0)),
        out_shape=jax.ShapeDtypeStruct((_N, 1), jnp.float32),
    )(h, agg0, agg1, deg0, deg1, cwhi, cwlo, conv_b, swhi, swlo, skip_b,
      w1hi, w1lo, dec_b1, dec_W2, dec_b2)


def kernel(x, edge_index, enc_W1, enc_b1, enc_W2, enc_b2, ln_g, ln_b,
           conv_W, conv_b, skip_W, skip_b, dec_W1, dec_b1, dec_W2, dec_b2):
    row = edge_index[0]
    col = edge_index[1]
    pk3 = (row | (col << 16)).reshape(_NW, _EPT)

    deg_parts = _sc_degree(pk3)
    deg0 = deg_parts[0].reshape(_NP, 1)
    deg1 = deg_parts[1].reshape(_NP, 1)
    w1hi, w1lo = _split_w(enc_W1)
    w2hi, w2lo = _split_w(enc_W2)
    h, p = _tc_encode_scale(x, w1hi, w1lo, enc_b1.reshape(1, -1), w2hi, w2lo,
                            enc_b2.reshape(1, -1), ln_g.reshape(1, -1),
                            ln_b.reshape(1, -1), deg0, deg1)
    agg_parts = _sc_aggregate(p, pk3)
    cwhi, cwlo = _split_w(conv_W)
    swhi, swlo = _split_w(skip_W)
    d1hi, d1lo = _split_w(dec_W1)
    return _tc_decode(h, agg_parts[0], agg_parts[1], deg0, deg1,
                      cwhi, cwlo, conv_b.reshape(1, -1), swhi, swlo,
                      skip_b.reshape(1, -1), d1hi, d1lo,
                      dec_b1.reshape(1, -1), dec_W2,
                      dec_b2.reshape(1, -1))


# speedup vs baseline: 1.1869x; 1.0166x over previous
import dataclasses
import functools

import jax
import jax.numpy as jnp
from jax import lax
from jax.experimental import pallas as pl
from jax.experimental.pallas import tpu as pltpu
from jax.experimental.pallas import tpu_sc as plsc

_N = 10000
_E = 320000
_NSUB = 16
_NW = 32
_EPT = _E // _NW
_CH = 80
_NCH = _EPT // _CH
_NP = 10240
_RPT = _NP // _NSUB

_BLK = 1000


def _sc_mesh():
    return plsc.VectorSubcoreMesh(core_axis_name="c", subcore_axis_name="s")


def _sc_degree(row2, col2):
    cp = pltpu.CompilerParams()
    if "needs_layout_passes" in pltpu.CompilerParams.__dataclass_fields__:
        cp = dataclasses.replace(cp, needs_layout_passes=False)

    @functools.partial(
        pl.kernel,
        out_type=[jax.ShapeDtypeStruct((2, _NP, 16), jnp.float32),
                  jax.ShapeDtypeStruct((_NW, _EPT), jnp.int32)],
        mesh=_sc_mesh(),
        compiler_params=cp,
        scratch_types=[
            pltpu.VMEM((_EPT,), jnp.int32),
            pltpu.VMEM((_EPT,), jnp.int32),
            pltpu.VMEM((_NP,), jnp.float32),
            pltpu.VMEM((_NSUB, _RPT // 5), jnp.float32),
            pltpu.VMEM((_RPT // 5,), jnp.float32),
            pltpu.VMEM((_RPT // 5, 16), jnp.float32),
            pltpu.VMEM_SHARED((_NSUB, _NP), jnp.float32),
        ],
    )
    def deg_kernel(row_hbm, col_hbm, out_hbm, pk_hbm,
                   ri_v, ci_v, tab_v, red_v, sum_v, deg16_v, stage_sh):
        c = lax.axis_index("c")
        s = lax.axis_index("s")
        wid = c * _NSUB + s
        pltpu.sync_copy(row_hbm.at[wid], ri_v)
        pltpu.sync_copy(col_hbm.at[wid], ci_v)

        @pl.loop(0, _NP // 16, unroll=8)
        def _(t):
            tab_v[pl.ds(t * 16, 16)] = jnp.zeros((16,), jnp.float32)

        ones = jnp.ones((16,), jnp.float32)

        @pl.loop(0, _EPT // 16, unroll=8)
        def _(n):
            ci = ci_v[pl.ds(n * 16, 16)]
            plsc.addupdate_scatter(tab_v, [ci], ones)
            ri_v[pl.ds(n * 16, 16)] = lax.bitwise_or(
                ri_v[pl.ds(n * 16, 16)], lax.shift_left(ci, 16))

        pltpu.sync_copy(tab_v, stage_sh.at[s])
        pltpu.sync_copy(ri_v, pk_hbm.at[wid])
        plsc.subcore_barrier()
        rc_n = _RPT // 5

        @pl.loop(0, 5)
        def _(rc):
            pltpu.sync_copy(
                stage_sh.at[:, pl.ds(s * _RPT + rc * rc_n, rc_n)], red_v)

            @pl.loop(0, rc_n // 16, unroll=4)
            def _(q):
                acc = red_v[0, pl.ds(q * 16, 16)]
                for r in range(1, _NSUB):
                    acc = acc + red_v[r, pl.ds(q * 16, 16)]
                sum_v[pl.ds(q * 16, 16)] = acc
                for r in range(16):
                    deg16_v[q * 16 + r, :] = plsc.load_gather(
                        sum_v, [jnp.full((16,), q * 16 + r, jnp.int32)])

            pltpu.sync_copy(deg16_v,
                            out_hbm.at[c, pl.ds(s * _RPT + rc * rc_n, rc_n)])

    return deg_kernel(row2, col2)


def _sc_aggregate(p, pk3):
    zeros = jnp.zeros((_RPT, 128), jnp.float32)

    @functools.partial(
        pl.kernel,
        out_type=jax.ShapeDtypeStruct((2, _NP, 128), jnp.float32),
        mesh=_sc_mesh(),
        scratch_types=[
            pltpu.VMEM((_EPT,), jnp.int32),
            pltpu.VMEM((_CH,), jnp.int32),
            pltpu.VMEM((_CH,), jnp.int32),
            pltpu.VMEM((_CH,), jnp.int32),
            pltpu.VMEM((_CH,), jnp.int32),
            pltpu.VMEM((_CH, 128), jnp.float32),
            pltpu.VMEM((_CH, 128), jnp.float32),
            pltpu.VMEM_SHARED((_NP, 128), jnp.float32),
            pltpu.SemaphoreType.DMA,
            pltpu.SemaphoreType.DMA,
            pltpu.SemaphoreType.DMA,
            pltpu.SemaphoreType.DMA,
        ],
    )
    def agg_kernel(p_hbm, pk_hbm, zeros_hbm, out_hbm,
                   pk_v, ri0_v, ci0_v, ri1_v, ci1_v, rows0_v, rows1_v,
                   agg_sh, gsem0, gsem1, ssem0, ssem1):
        c = lax.axis_index("c")
        s = lax.axis_index("s")
        wid = c * _NSUB + s
        pltpu.sync_copy(pk_hbm.at[wid], pk_v)
        pltpu.sync_copy(zeros_hbm, agg_sh.at[pl.ds(s * _RPT, _RPT)])

        def unpack(m, ri_b, ci_b):
            @pl.loop(0, _CH // 16)
            def _(t):
                v = pk_v[pl.ds(m * _CH + t * 16, 16)]
                ri_b[pl.ds(t * 16, 16)] = lax.bitwise_and(v, 0xFFFF)
                ci_b[pl.ds(t * 16, 16)] = lax.shift_right_logical(v, 16)

        unpack(0, ri0_v, ci0_v)
        unpack(1, ri1_v, ci1_v)
        plsc.subcore_barrier()

        pltpu.async_copy(p_hbm.at[ri0_v], rows0_v, gsem0)
        pltpu.async_copy(p_hbm.at[ri1_v], rows1_v, gsem1)

        @pl.loop(0, (_NCH - 1) // 2)
        def _(k):
            j1 = 2 * k + 1
            pltpu.make_async_copy(p_hbm.at[ri0_v], rows0_v, gsem0).wait()
            pltpu.async_copy(rows0_v, agg_sh.at[ci0_v], ssem0, add=True)
            pltpu.make_async_copy(p_hbm.at[ri1_v], rows1_v, gsem1).wait()
            pltpu.async_copy(rows1_v, agg_sh.at[ci1_v], ssem1, add=True)
            pltpu.make_async_copy(rows0_v, agg_sh.at[ci0_v], ssem0).wait()
            unpack(j1 + 1, ri0_v, ci0_v)
            pltpu.async_copy(p_hbm.at[ri0_v], rows0_v, gsem0)
            pltpu.make_async_copy(rows1_v, agg_sh.at[ci1_v], ssem1).wait()

            @pl.when(j1 + 2 < _NCH)
            def _():
                unpack(j1 + 2, ri1_v, ci1_v)
                pltpu.async_copy(p_hbm.at[ri1_v], rows1_v, gsem1)

        pltpu.make_async_copy(p_hbm.at[ri0_v], rows0_v, gsem0).wait()
        pltpu.sync_copy(rows0_v, agg_sh.at[ci0_v], add=True)

        plsc.subcore_barrier()
        pltpu.sync_copy(agg_sh.at[pl.ds(s * _RPT, _RPT)],
                        out_hbm.at[c, pl.ds(s * _RPT, _RPT)])

    return agg_kernel(p, pk3, zeros)


def _dot3(x, whi, wlo):
    xhi = x.astype(jnp.bfloat16)
    xlo = (x - xhi.astype(jnp.float32)).astype(jnp.bfloat16)
    return (jnp.dot(xhi, whi, preferred_element_type=jnp.float32)
            + jnp.dot(xhi, wlo, preferred_element_type=jnp.float32)
            + jnp.dot(xlo, whi, preferred_element_type=jnp.float32))


def _split_w(w):
    whi = w.astype(jnp.bfloat16)
    wlo = (w - whi.astype(jnp.float32)).astype(jnp.bfloat16)
    return whi, wlo


def _tc_encode_scale(x, w1hi, w1lo, b1, w2hi, w2lo, b2, g, b, deg0, deg1):
    def body(x_ref, w1hi_ref, w1lo_ref, b1_ref, w2hi_ref, w2lo_ref, b2_ref,
             g_ref, bb_ref, d0_ref, d1_ref, h_ref, p_ref):
        h = jnp.maximum(
            _dot3(x_ref[...], w1hi_ref[...], w1lo_ref[...]) + b1_ref[...], 0.0)
        h = _dot3(h, w2hi_ref[...], w2lo_ref[...]) + b2_ref[...]
        mu = jnp.mean(h, axis=-1, keepdims=True)
        var = jnp.mean((h - mu) ** 2, axis=-1, keepdims=True)
        h = (h - mu) * lax.rsqrt(var + 1e-5) * g_ref[...] + bb_ref[...]
        h_ref[...] = h
        d = d0_ref[0][:, :1] + d1_ref[0][:, :1] + 2.0
        p_ref[...] = lax.rsqrt(d) * h

    full = lambda shape: pl.BlockSpec(shape, lambda i: (0, 0))
    return pl.pallas_call(
        body,
        grid=(_N // _BLK,),
        in_specs=[
            pl.BlockSpec((_BLK, 128), lambda i: (i, 0)),
            full((128, 128)), full((128, 128)), full((1, 128)),
            full((128, 128)), full((128, 128)), full((1, 128)),
            full((1, 128)), full((1, 128)),
            pl.BlockSpec((1, _BLK, 16), lambda i: (0, i, 0)),
            pl.BlockSpec((1, _BLK, 16), lambda i: (1, i, 0)),
        ],
        out_specs=[pl.BlockSpec((_BLK, 128), lambda i: (i, 0)),
                   pl.BlockSpec((_BLK, 128), lambda i: (i, 0))],
        out_shape=[jax.ShapeDtypeStruct((_N, 128), jnp.float32),
                   jax.ShapeDtypeStruct((_N, 128), jnp.float32)],
    )(x, w1hi, w1lo, b1, w2hi, w2lo, b2, g, b, deg0, deg1)


def _tc_decode(h, agg0, agg1, deg0, deg1, cwhi, cwlo, conv_b, swhi, swlo,
               skip_b, w1hi, w1lo, dec_b1, dec_W2, dec_b2):
    def body(h_ref, a0_ref, a1_ref, d0_ref, d1_ref, cwhi_ref, cwlo_ref,
             cb_ref, swhi_ref, swlo_ref, sb_ref, w1hi_ref, w1lo_ref, b1_ref,
             w2_ref, b2_ref, y_ref):
        h = h_ref[...]
        d = d0_ref[0][:, :1] + d1_ref[0][:, :1] + 2.0
        dinv = lax.rsqrt(d)
        tmp = dinv * (a0_ref[0] + a1_ref[0]) + (2.0 * dinv * dinv) * h
        out = (_dot3(tmp, cwhi_ref[...], cwlo_ref[...]) + cb_ref[...]
               + _dot3(h, swhi_ref[...], swlo_ref[...]) + sb_ref[...])
        out = jnp.where(out > 0, out, 0.1 * (jnp.exp(out) - 1.0))
        dd = _dot3(out, w1hi_ref[...], w1lo_ref[...]) + b1_ref[...]
        dd = jnp.where(dd > 0, dd, 0.1 * dd)
        y_ref[...] = jnp.dot(dd, w2_ref[...],
                             preferred_element_type=jnp.float32,
                    precision=lax.Precision.HIGHEST) + b2_ref[...]

    full = lambda shape: pl.BlockSpec(shape, lambda i: (0, 0))
    return pl.pallas_call(
        body,
        grid=(_N // _BLK,),
        in_specs=[
            pl.BlockSpec((_BLK, 128), lambda i: (i, 0)),
            pl.BlockSpec((1, _BLK, 128), lambda i: (0, i, 0)),
            pl.BlockSpec((1, _BLK, 128), lambda i: (1, i, 0)),
            pl.BlockSpec((1, _BLK, 16), lambda i: (0, i, 0)),
            pl.BlockSpec((1, _BLK, 16), lambda i: (1, i, 0)),
            full((128, 256)), full((128, 256)), full((1, 256)),
            full((128, 256)), full((128, 256)), full((1, 256)),
            full((256, 256)), full((256, 256)), full((1, 256)),
            full((256, 1)), full((1, 1)),
        ],
        out_specs=pl.BlockSpec((_BLK, 1), lambda i: (i, 0)),
        out_shape=jax.ShapeDtypeStruct((_N, 1), jnp.float32),
    )(h, agg0, agg1, deg0, deg1, cwhi, cwlo, conv_b, swhi, swlo, skip_b,
      w1hi, w1lo, dec_b1, dec_W2, dec_b2)


def kernel(x, edge_index, enc_W1, enc_b1, enc_W2, enc_b2, ln_g, ln_b,
           conv_W, conv_b, skip_W, skip_b, dec_W1, dec_b1, dec_W2, dec_b2):
    row2 = edge_index[0].reshape(_NW, _EPT)
    col2 = edge_index[1].reshape(_NW, _EPT)

    deg_parts, pk3 = _sc_degree(row2, col2)
    w1hi, w1lo = _split_w(enc_W1)
    w2hi, w2lo = _split_w(enc_W2)
    h, p = _tc_encode_scale(x, w1hi, w1lo, enc_b1.reshape(1, -1), w2hi, w2lo,
                            enc_b2.reshape(1, -1), ln_g.reshape(1, -1),
                            ln_b.reshape(1, -1), deg_parts, deg_parts)
    agg_parts = _sc_aggregate(p, pk3)
    cwhi, cwlo = _split_w(conv_W)
    swhi, swlo = _split_w(skip_W)
    d1hi, d1lo = _split_w(dec_W1)
    return _tc_decode(h, agg_parts, agg_parts, deg_parts, deg_parts,
                      cwhi, cwlo, conv_b.reshape(1, -1), swhi, swlo,
                      skip_b.reshape(1, -1), d1hi, d1lo,
                      dec_b1.reshape(1, -1), dec_W2,
                      dec_b2.reshape(1, -1))


# speedup vs baseline: 1.2291x; 1.0356x over previous
import dataclasses
import functools

import jax
import jax.numpy as jnp
from jax import lax
from jax.experimental import pallas as pl
from jax.experimental.pallas import tpu as pltpu
from jax.experimental.pallas import tpu_sc as plsc

_N = 10000
_E = 320000
_NSUB = 16
_NW = 32
_EPT = _E // _NW
_CH = 80
_NCH = _EPT // _CH
_NP = 10240
_RPT = _NP // _NSUB

_BLK = 1000


def _sc_mesh():
    return plsc.VectorSubcoreMesh(core_axis_name="c", subcore_axis_name="s")


def _sc_degree(ei4):
    cp = pltpu.CompilerParams()
    if "needs_layout_passes" in pltpu.CompilerParams.__dataclass_fields__:
        cp = dataclasses.replace(cp, needs_layout_passes=False)

    @functools.partial(
        pl.kernel,
        out_type=[jax.ShapeDtypeStruct((2, _NP, 16), jnp.float32),
                  jax.ShapeDtypeStruct((_NW, _NCH, _CH), jnp.int32)],
        mesh=_sc_mesh(),
        compiler_params=cp,
        scratch_types=[
            pltpu.VMEM((_NCH, _CH), jnp.int32),
            pltpu.VMEM((_NCH, _CH), jnp.int32),
            pltpu.VMEM((_NP,), jnp.float32),
            pltpu.VMEM((_NSUB, _RPT // 5), jnp.float32),
            pltpu.VMEM((_RPT // 5,), jnp.float32),
            pltpu.VMEM((_RPT // 5, 16), jnp.float32),
            pltpu.VMEM_SHARED((_NSUB, _NP), jnp.float32),
        ],
    )
    def deg_kernel(ei_hbm, out_hbm, pk_hbm,
                   ri_v, ci_v, tab_v, red_v, sum_v, deg16_v, stage_sh):
        c = lax.axis_index("c")
        s = lax.axis_index("s")
        wid = c * _NSUB + s
        pltpu.sync_copy(ei_hbm.at[0, wid], ri_v)
        pltpu.sync_copy(ei_hbm.at[1, wid], ci_v)

        @pl.loop(0, _NP // 16, unroll=8)
        def _(t):
            tab_v[pl.ds(t * 16, 16)] = jnp.zeros((16,), jnp.float32)

        ones = jnp.ones((16,), jnp.float32)

        @pl.loop(0, _NCH, unroll=2)
        def _(m):
            @pl.loop(0, _CH // 16, unroll=5)
            def _(t):
                ci = ci_v.at[m][pl.ds(t * 16, 16)]
                plsc.addupdate_scatter(tab_v, [ci], ones)
                ri_v.at[m][pl.ds(t * 16, 16)] = lax.bitwise_or(
                    ri_v.at[m][pl.ds(t * 16, 16)], lax.shift_left(ci, 16))

        pltpu.sync_copy(tab_v, stage_sh.at[s])
        pltpu.sync_copy(ri_v, pk_hbm.at[wid])
        plsc.subcore_barrier()
        rc_n = _RPT // 5

        @pl.loop(0, 5)
        def _(rc):
            pltpu.sync_copy(
                stage_sh.at[:, pl.ds(s * _RPT + rc * rc_n, rc_n)], red_v)

            @pl.loop(0, rc_n // 16, unroll=4)
            def _(q):
                acc = red_v[0, pl.ds(q * 16, 16)]
                for r in range(1, _NSUB):
                    acc = acc + red_v[r, pl.ds(q * 16, 16)]
                sum_v[pl.ds(q * 16, 16)] = acc
                for r in range(16):
                    deg16_v[q * 16 + r, :] = plsc.load_gather(
                        sum_v, [jnp.full((16,), q * 16 + r, jnp.int32)])

            pltpu.sync_copy(deg16_v,
                            out_hbm.at[c, pl.ds(s * _RPT + rc * rc_n, rc_n)])

    return deg_kernel(ei4)


def _sc_aggregate(p, pk3):
    zeros = jnp.zeros((_RPT, 128), jnp.float32)

    @functools.partial(
        pl.kernel,
        out_type=jax.ShapeDtypeStruct((2, _NP, 128), jnp.float32),
        mesh=_sc_mesh(),
        scratch_types=[
            pltpu.VMEM((_NCH, _CH), jnp.int32),
            pltpu.VMEM((_CH,), jnp.int32),
            pltpu.VMEM((_CH,), jnp.int32),
            pltpu.VMEM((_CH,), jnp.int32),
            pltpu.VMEM((_CH,), jnp.int32),
            pltpu.VMEM((_CH, 128), jnp.float32),
            pltpu.VMEM((_CH, 128), jnp.float32),
            pltpu.VMEM_SHARED((_NP, 128), jnp.float32),
            pltpu.SemaphoreType.DMA,
            pltpu.SemaphoreType.DMA,
            pltpu.SemaphoreType.DMA,
            pltpu.SemaphoreType.DMA,
        ],
    )
    def agg_kernel(p_hbm, pk_hbm, zeros_hbm, out_hbm,
                   pk_v, ri0_v, ci0_v, ri1_v, ci1_v, rows0_v, rows1_v,
                   agg_sh, gsem0, gsem1, ssem0, ssem1):
        c = lax.axis_index("c")
        s = lax.axis_index("s")
        wid = c * _NSUB + s
        pltpu.sync_copy(pk_hbm.at[wid], pk_v)
        pltpu.sync_copy(zeros_hbm, agg_sh.at[pl.ds(s * _RPT, _RPT)])

        def unpack(m, ri_b, ci_b):
            @pl.loop(0, _CH // 16)
            def _(t):
                v = pk_v.at[m][pl.ds(t * 16, 16)]
                ri_b[pl.ds(t * 16, 16)] = lax.bitwise_and(v, 0xFFFF)
                ci_b[pl.ds(t * 16, 16)] = lax.shift_right_logical(v, 16)

        unpack(0, ri0_v, ci0_v)
        unpack(1, ri1_v, ci1_v)
        plsc.subcore_barrier()

        pltpu.async_copy(p_hbm.at[ri0_v], rows0_v, gsem0)
        pltpu.async_copy(p_hbm.at[ri1_v], rows1_v, gsem1)

        @pl.loop(0, (_NCH - 1) // 2)
        def _(k):
            j1 = 2 * k + 1
            pltpu.make_async_copy(p_hbm.at[ri0_v], rows0_v, gsem0).wait()
            pltpu.async_copy(rows0_v, agg_sh.at[ci0_v], ssem0, add=True)
            pltpu.make_async_copy(p_hbm.at[ri1_v], rows1_v, gsem1).wait()
            pltpu.async_copy(rows1_v, agg_sh.at[ci1_v], ssem1, add=True)
            pltpu.make_async_copy(rows0_v, agg_sh.at[ci0_v], ssem0).wait()
            unpack(j1 + 1, ri0_v, ci0_v)
            pltpu.async_copy(p_hbm.at[ri0_v], rows0_v, gsem0)
            pltpu.make_async_copy(rows1_v, agg_sh.at[ci1_v], ssem1).wait()

            @pl.when(j1 + 2 < _NCH)
            def _():
                unpack(j1 + 2, ri1_v, ci1_v)
                pltpu.async_copy(p_hbm.at[ri1_v], rows1_v, gsem1)

        pltpu.make_async_copy(p_hbm.at[ri0_v], rows0_v, gsem0).wait()
        pltpu.sync_copy(rows0_v, agg_sh.at[ci0_v], add=True)

        plsc.subcore_barrier()
        pltpu.sync_copy(agg_sh.at[pl.ds(s * _RPT, _RPT)],
                        out_hbm.at[c, pl.ds(s * _RPT, _RPT)])

    return agg_kernel(p, pk3, zeros)


def _dot3(x, whi, wlo):
    xhi = x.astype(jnp.bfloat16)
    xlo = (x - xhi.astype(jnp.float32)).astype(jnp.bfloat16)
    return (jnp.dot(xhi, whi, preferred_element_type=jnp.float32)
            + jnp.dot(xhi, wlo, preferred_element_type=jnp.float32)
            + jnp.dot(xlo, whi, preferred_element_type=jnp.float32))


def _split_w(w):
    whi = w.astype(jnp.bfloat16)
    wlo = (w - whi.astype(jnp.float32)).astype(jnp.bfloat16)
    return whi, wlo


def _tc_encode_scale(x, w1hi, w1lo, b1, w2hi, w2lo, b2, g, b, deg0, deg1):
    def body(x_ref, w1hi_ref, w1lo_ref, b1_ref, w2hi_ref, w2lo_ref, b2_ref,
             g_ref, bb_ref, d0_ref, d1_ref, h_ref, p_ref):
        h = jnp.maximum(
            _dot3(x_ref[...], w1hi_ref[...], w1lo_ref[...]) + b1_ref[...], 0.0)
        h = _dot3(h, w2hi_ref[...], w2lo_ref[...]) + b2_ref[...]
        mu = jnp.mean(h, axis=-1, keepdims=True)
        var = jnp.mean((h - mu) ** 2, axis=-1, keepdims=True)
        h = (h - mu) * lax.rsqrt(var + 1e-5) * g_ref[...] + bb_ref[...]
        h_ref[...] = h
        d = d0_ref[0][:, :1] + d1_ref[0][:, :1] + 2.0
        p_ref[...] = lax.rsqrt(d) * h

    full = lambda shape: pl.BlockSpec(shape, lambda i: (0, 0))
    return pl.pallas_call(
        body,
        grid=(_N // _BLK,),
        in_specs=[
            pl.BlockSpec((_BLK, 128), lambda i: (i, 0)),
            full((128, 128)), full((128, 128)), full((1, 128)),
            full((128, 128)), full((128, 128)), full((1, 128)),
            full((1, 128)), full((1, 128)),
            pl.BlockSpec((1, _BLK, 16), lambda i: (0, i, 0)),
            pl.BlockSpec((1, _BLK, 16), lambda i: (1, i, 0)),
        ],
        out_specs=[pl.BlockSpec((_BLK, 128), lambda i: (i, 0)),
                   pl.BlockSpec((_BLK, 128), lambda i: (i, 0))],
        out_shape=[jax.ShapeDtypeStruct((_N, 128), jnp.float32),
                   jax.ShapeDtypeStruct((_N, 128), jnp.float32)],
    )(x, w1hi, w1lo, b1, w2hi, w2lo, b2, g, b, deg0, deg1)


def _tc_decode(h, agg0, agg1, deg0, deg1, cwhi, cwlo, conv_b, swhi, swlo,
               skip_b, w1hi, w1lo, dec_b1, dec_W2, dec_b2):
    def body(h_ref, a0_ref, a1_ref, d0_ref, d1_ref, cwhi_ref, cwlo_ref,
             cb_ref, swhi_ref, swlo_ref, sb_ref, w1hi_ref, w1lo_ref, b1_ref,
             w2_ref, b2_ref, y_ref):
        h = h_ref[...]
        d = d0_ref[0][:, :1] + d1_ref[0][:, :1] + 2.0
        dinv = lax.rsqrt(d)
        tmp = dinv * (a0_ref[0] + a1_ref[0]) + (2.0 * dinv * dinv) * h
        out = (_dot3(tmp, cwhi_ref[...], cwlo_ref[...]) + cb_ref[...]
               + _dot3(h, swhi_ref[...], swlo_ref[...]) + sb_ref[...])
        out = jnp.where(out > 0, out, 0.1 * (jnp.exp(out) - 1.0))
        dd = _dot3(out, w1hi_ref[...], w1lo_ref[...]) + b1_ref[...]
        dd = jnp.where(dd > 0, dd, 0.1 * dd)
        y_ref[...] = jnp.dot(dd, w2_ref[...],
                             preferred_element_type=jnp.float32,
                    precision=lax.Precision.HIGHEST) + b2_ref[...]

    full = lambda shape: pl.BlockSpec(shape, lambda i: (0, 0))
    return pl.pallas_call(
        body,
        grid=(_N // _BLK,),
        in_specs=[
            pl.BlockSpec((_BLK, 128), lambda i: (i, 0)),
            pl.BlockSpec((1, _BLK, 128), lambda i: (0, i, 0)),
            pl.BlockSpec((1, _BLK, 128), lambda i: (1, i, 0)),
            pl.BlockSpec((1, _BLK, 16), lambda i: (0, i, 0)),
            pl.BlockSpec((1, _BLK, 16), lambda i: (1, i, 0)),
            full((128, 256)), full((128, 256)), full((1, 256)),
            full((128, 256)), full((128, 256)), full((1, 256)),
            full((256, 256)), full((256, 256)), full((1, 256)),
            full((256, 1)), full((1, 1)),
        ],
        out_specs=pl.BlockSpec((_BLK, 1), lambda i: (i, 0)),
        out_shape=jax.ShapeDtypeStruct((_N, 1), jnp.float32),
    )(h, agg0, agg1, deg0, deg1, cwhi, cwlo, conv_b, swhi, swlo, skip_b,
      w1hi, w1lo, dec_b1, dec_W2, dec_b2)


def kernel(x, edge_index, enc_W1, enc_b1, enc_W2, enc_b2, ln_g, ln_b,
           conv_W, conv_b, skip_W, skip_b, dec_W1, dec_b1, dec_W2, dec_b2):
    ei4 = edge_index.reshape(2, _NW, _NCH, _CH)

    deg_parts, pk3 = _sc_degree(ei4)
    w1hi, w1lo = _split_w(enc_W1)
    w2hi, w2lo = _split_w(enc_W2)
    h, p = _tc_encode_scale(x, w1hi, w1lo, enc_b1.reshape(1, -1), w2hi, w2lo,
                            enc_b2.reshape(1, -1), ln_g.reshape(1, -1),
                            ln_b.reshape(1, -1), deg_parts, deg_parts)
    agg_parts = _sc_aggregate(p, pk3)
    cwhi, cwlo = _split_w(conv_W)
    swhi, swlo = _split_w(skip_W)
    d1hi, d1lo = _split_w(dec_W1)
    return _tc_decode(h, agg_parts, agg_parts, deg_parts, deg_parts,
                      cwhi, cwlo, conv_b.reshape(1, -1), swhi, swlo,
                      skip_b.reshape(1, -1), d1hi, d1lo,
                      dec_b1.reshape(1, -1), dec_W2,
                      dec_b2.reshape(1, -1))
